# baseline (speedup 1.0000x reference)
import jax
import jax.numpy as jnp
from jax import lax
from jax.experimental import pallas as pl
from jax.experimental.pallas import tpu as pltpu
from jax.experimental.pallas import tpu_sc as plsc

_NODE_NUM = 100000
_REL_NUM = 1000
_D = 128
_K = 20
_P = 16384
_NS = 2 * _P
_NE = 2 * _NS
_MARGIN = 1.0

_NW = 32
_HG = 8
_NHG = _NE // _HG
_HGPW = _NHG // _NW
_NROWS = _HG * _K
_GROWS = _NROWS + _HG
_STRIDE = 176
_IBATCH = 64
_NBATCH = _HGPW // _IBATCH
_NSLOT = 4
_RPW = _NS // _NW
_RCHUNKS = _RPW // 128


def _sc_body(idx5_hbm, relidx_hbm, node_hbm, rel_hbm,
             out_enc, out_rel,
             idx0, idx1, nb0, nb1, nb2, nb3, ob0, ob1, ob2, ob3, relidx_v,
             sg0, sg1, sg2, sg3, so0, so1, so2, so3, semI, semR0, semR1):
    c = lax.axis_index("c")
    s = lax.axis_index("s")
    w = s * 2 + c
    hbase = w * _HGPW
    ibase = w * (_HGPW * _STRIDE)

    nbufs = [nb0, nb1, nb2, nb3]
    obufs = [ob0, ob1, ob2, ob3]
    gsems = [sg0, sg1, sg2, sg3]
    osems = [so0, so1, so2, so3]

    def stage_idx(bi, ibuf):
        return pltpu.async_copy(
            idx5_hbm.at[pl.ds(ibase + bi * (_IBATCH * _STRIDE),
                              _IBATCH * _STRIDE)], ibuf, semI)

    def issue(H, k):
        off = lax.rem(H, _IBATCH) * _STRIDE
        par = lax.rem(lax.div(H, _IBATCH), 2)

        @pl.when(par == 0)
        def _():
            pltpu.async_copy(node_hbm.at[idx0.at[pl.ds(off, _GROWS)]],
                             nbufs[k].at[pl.ds(0, _GROWS)], gsems[k])

        @pl.when(par == 1)
        def _():
            pltpu.async_copy(node_hbm.at[idx1.at[pl.ds(off, _GROWS)]],
                             nbufs[k].at[pl.ds(0, _GROWS)], gsems[k])

    def wait_gather(k):
        pltpu.make_async_copy(node_hbm.at[idx0.at[pl.ds(0, _GROWS)]],
                              nbufs[k].at[pl.ds(0, _GROWS)], gsems[k]).wait()

    def wait_out(k):
        pltpu.make_async_copy(obufs[k], out_enc.at[pl.ds(0, _HG)],
                              osems[k]).wait()

    def sum_group(k):
        nbuf, obuf = nbufs[k], obufs[k]

        def sample_body(i, carry2):
            base = i * _K
            acc = tuple(nbuf[base, pl.ds(j * 16, 16)] for j in range(8))

            def kbody(kk, accs):
                return tuple(accs[j] + nbuf[base + kk, pl.ds(j * 16, 16)]
                             for j in range(8))

            acc = lax.fori_loop(1, _K, kbody, acc)
            for j in range(8):
                obuf[i, pl.ds(_D + j * 16, 16)] = acc[j]
                obuf[i, pl.ds(j * 16, 16)] = nbuf[_NROWS + i,
                                                  pl.ds(j * 16, 16)]
            return carry2

        lax.fori_loop(0, _HG, sample_body, 0)

    stage_idx(0, idx0).wait()
    stage_idx(1, idx1)
    for k in range(_NSLOT - 1):
        issue(jnp.int32(k), k)

    def ring_body(t, carry):
        for k in range(_NSLOT):
            hcons = _NSLOT * t + k
            wait_gather(k)

            @pl.when(t > 0)
            def _():
                wait_out(k)

            sum_group(k)
            pltpu.async_copy(
                obufs[k],
                out_enc.at[pl.ds((hbase + hcons) * _HG, _HG)], osems[k])

            hpre = hcons + (_NSLOT - 1)
            mod = lax.rem(hpre, _IBATCH)
            bb = lax.div(hpre, _IBATCH)

            @pl.when(jnp.logical_and(mod == 0, hpre < _HGPW))
            def _():
                pltpu.make_async_copy(
                    idx5_hbm.at[pl.ds(0, _IBATCH * _STRIDE)], idx0,
                    semI).wait()

            @pl.when(jnp.logical_and(mod == 2, bb + 1 <= _NBATCH - 1))
            def _():
                @pl.when(lax.rem(bb + 1, 2) == 0)
                def _():
                    stage_idx(bb + 1, idx0)

                @pl.when(lax.rem(bb + 1, 2) == 1)
                def _():
                    stage_idx(bb + 1, idx1)

            @pl.when(hpre < _HGPW)
            def _():
                issue(hpre, (k + _NSLOT - 1) % _NSLOT)
        return carry

    lax.fori_loop(0, _HGPW // _NSLOT, ring_body, 0)
    for k in range(_NSLOT):
        wait_out(k)

    pltpu.async_copy(relidx_hbm.at[pl.ds(w * _RCHUNKS, _RCHUNKS)],
                     relidx_v, semI).wait()
    pltpu.async_copy(rel_hbm.at[relidx_v.at[0]],
                     nb0.at[pl.ds(0, 128)], semR0)

    def rel_chunk(t, carry):
        @pl.when(t + 1 < _RCHUNKS)
        def _():
            @pl.when(lax.rem(t, 2) == 0)
            def _():
                pltpu.async_copy(rel_hbm.at[relidx_v.at[t + 1]],
                                 nb1.at[pl.ds(0, 128)], semR1)

            @pl.when(lax.rem(t, 2) == 1)
            def _():
                pltpu.async_copy(rel_hbm.at[relidx_v.at[t + 1]],
                                 nb0.at[pl.ds(0, 128)], semR0)

        @pl.when(lax.rem(t, 2) == 0)
        def _():
            pltpu.make_async_copy(rel_hbm.at[relidx_v.at[0]],
                                  nb0.at[pl.ds(0, 128)], semR0).wait()
            pltpu.sync_copy(nb0.at[pl.ds(0, 128)],
                            out_rel.at[pl.ds(w * _RPW + t * 128, 128)])

        @pl.when(lax.rem(t, 2) == 1)
        def _():
            pltpu.make_async_copy(rel_hbm.at[relidx_v.at[0]],
                                  nb1.at[pl.ds(0, 128)], semR1).wait()
            pltpu.sync_copy(nb1.at[pl.ds(0, 128)],
                            out_rel.at[pl.ds(w * _RPW + t * 128, 128)])
        return carry

    lax.fori_loop(0, _RCHUNKS, rel_chunk, 0)


def _sc_gather(idx5, relidx, node_emb, rel_emb):
    mesh = plsc.VectorSubcoreMesh(core_axis_name="c", subcore_axis_name="s")
    return pl.kernel(
        _sc_body,
        out_type=(
            jax.ShapeDtypeStruct((_NE, 2 * _D), jnp.float32),
            jax.ShapeDtypeStruct((_NS, _D), jnp.float32),
        ),
        mesh=mesh,
        scratch_types=[
            pltpu.VMEM((_IBATCH * _STRIDE,), jnp.int32),
            pltpu.VMEM((_IBATCH * _STRIDE,), jnp.int32),
            pltpu.VMEM((_GROWS, _D), jnp.float32),
            pltpu.VMEM((_GROWS, _D), jnp.float32),
            pltpu.VMEM((_GROWS, _D), jnp.float32),
            pltpu.VMEM((_GROWS, _D), jnp.float32),
            pltpu.VMEM((_HG, 2 * _D), jnp.float32),
            pltpu.VMEM((_HG, 2 * _D), jnp.float32),
            pltpu.VMEM((_HG, 2 * _D), jnp.float32),
            pltpu.VMEM((_HG, 2 * _D), jnp.float32),
            pltpu.VMEM((_RCHUNKS, 128), jnp.int32),
            pltpu.SemaphoreType.DMA,
            pltpu.SemaphoreType.DMA,
            pltpu.SemaphoreType.DMA,
            pltpu.SemaphoreType.DMA,
            pltpu.SemaphoreType.DMA,
            pltpu.SemaphoreType.DMA,
            pltpu.SemaphoreType.DMA,
            pltpu.SemaphoreType.DMA,
            pltpu.SemaphoreType.DMA,
            pltpu.SemaphoreType.DMA,
            pltpu.SemaphoreType.DMA,
        ],
    )(idx5, relidx, node_emb, rel_emb)


_BP = 4096
_NBLK = _P // _BP


def _tc_body(enc_sp, enc_sn, enc_dp, enc_dn,
             rel_p, rel_n, wf, bvec, out):
    i = pl.program_id(0)

    def enc(x):
        h = lax.dot_general(x[...], wf[...], (((1,), (0,)), ((), ())),
                            preferred_element_type=jnp.float32)
        return jnp.maximum(h + bvec[...], 0.0)

    hsp = enc(enc_sp)
    hsn = enc(enc_sn)
    hdp = enc(enc_dp)
    hdn = enc(enc_dn)
    dp = hsp + rel_p[...] - hdp
    dn = hsn + rel_n[...] - hdn
    sp = -jnp.sqrt(jnp.sum(dp * dp, axis=1) + 1e-12)
    sn = -jnp.sqrt(jnp.sum(dn * dn, axis=1) + 1e-12)
    part = jnp.sum(jnp.maximum(0.0, sn - sp + _MARGIN)) * (1.0 / _P)

    @pl.when(i == 0)
    def _():
        out[0, 0] = 0.0

    out[0, 0] += part


def _tc_dense(out_enc, out_rel, wf, bvec):
    enc_spec = lambda off: pl.BlockSpec((_BP, 2 * _D),
                                        lambda i, o=off: (i + o, 0))
    rel_spec = lambda off: pl.BlockSpec((_BP, _D),
                                        lambda i, o=off: (i + o, 0))
    loss = pl.pallas_call(
        _tc_body,
        grid=(_NBLK,),
        in_specs=[
            enc_spec(0), enc_spec(_NBLK), enc_spec(2 * _NBLK),
            enc_spec(3 * _NBLK),
            rel_spec(0), rel_spec(_NBLK),
            pl.BlockSpec((2 * _D, _D), lambda i: (0, 0)),
            pl.BlockSpec((1, _D), lambda i: (0, 0)),
        ],
        out_specs=pl.BlockSpec((1, 1), lambda i: (0, 0),
                               memory_space=pltpu.SMEM),
        out_shape=jax.ShapeDtypeStruct((1, 1), jnp.float32),
        compiler_params=pltpu.CompilerParams(
            dimension_semantics=("arbitrary",)),
    )(out_enc, out_enc, out_enc, out_enc,
      out_rel, out_rel, wf, bvec)
    return loss[0, 0]


def kernel(train_pos, train_neg, ngh_idx_src, ngh_idx_dst,
           node_emb, rel_emb, W, b):
    alls = jnp.concatenate([train_pos, train_neg], axis=0).astype(jnp.int32)
    src = alls[:, 0] % _NODE_NUM
    dst = alls[:, 1] % _NODE_NUM
    rel = alls[:, 2] % _REL_NUM

    selfidx = jnp.concatenate([src, dst]).reshape(_NHG, _HG)
    nghidx = jnp.concatenate(
        [ngh_idx_src, ngh_idx_dst], axis=0).astype(jnp.int32).reshape(
            _NHG, _NROWS)
    pad = jnp.zeros((_NHG, _STRIDE - _GROWS), dtype=jnp.int32)
    idx5 = jnp.concatenate([nghidx, selfidx, pad], axis=1).reshape(-1)
    relidx = rel.reshape(_NS // 128, 128)

    out_enc, out_rel = _sc_gather(idx5, relidx, node_emb, rel_emb)

    wf = jnp.concatenate([W[:_D], W[_D:] * (1.0 / _K)], axis=0)
    bvec = b.reshape(1, _D)
    return _tc_dense(out_enc, out_rel, wf, bvec)


# speedup vs baseline: 1.0019x; 1.0019x over previous
import jax
import jax.numpy as jnp
from jax import lax
from jax.experimental import pallas as pl
from jax.experimental.pallas import tpu as pltpu
from jax.experimental.pallas import tpu_sc as plsc

_NODE_NUM = 100000
_REL_NUM = 1000
_D = 128
_K = 20
_P = 16384
_NS = 2 * _P
_NE = 2 * _NS
_MARGIN = 1.0

_NW = 32
_HG = 8
_NHG = _NE // _HG
_HGPW = _NHG // _NW
_NROWS = _HG * _K
_GROWS = _NROWS + _HG
_STRIDE = 176
_IBATCH = 64
_NBATCH = _HGPW // _IBATCH
_NSLOT = 4
_RPW = _NS // _NW
_RCHUNKS = _RPW // 128


def _sc_body(idx5_hbm, relidx_hbm, node_hbm, rel_hbm,
             out_enc, out_rel,
             idx0, idx1, nb0, nb1, nb2, nb3, ob0, ob1, ob2, ob3, relidx_v,
             sg0, sg1, sg2, sg3, so0, so1, so2, so3, semI, semR0, semR1):
    c = lax.axis_index("c")
    s = lax.axis_index("s")
    w = s * 2 + c
    hbase = w * _HGPW
    ibase = w * (_HGPW * _STRIDE)

    nbufs = [nb0, nb1, nb2, nb3]
    obufs = [ob0, ob1, ob2, ob3]
    gsems = [sg0, sg1, sg2, sg3]
    osems = [so0, so1, so2, so3]

    def stage_idx(bi, ibuf):
        return pltpu.async_copy(
            idx5_hbm.at[pl.ds(ibase + bi * (_IBATCH * _STRIDE),
                              _IBATCH * _STRIDE)], ibuf, semI)

    def issue(H, k):
        off = lax.rem(H, _IBATCH) * _STRIDE
        par = lax.rem(lax.div(H, _IBATCH), 2)

        @pl.when(par == 0)
        def _():
            pltpu.async_copy(node_hbm.at[idx0.at[pl.ds(off, _GROWS)]],
                             nbufs[k].at[pl.ds(0, _GROWS)], gsems[k])

        @pl.when(par == 1)
        def _():
            pltpu.async_copy(node_hbm.at[idx1.at[pl.ds(off, _GROWS)]],
                             nbufs[k].at[pl.ds(0, _GROWS)], gsems[k])

    def wait_gather(k):
        pltpu.make_async_copy(node_hbm.at[idx0.at[pl.ds(0, _GROWS)]],
                              nbufs[k].at[pl.ds(0, _GROWS)], gsems[k]).wait()

    def wait_out(k):
        pltpu.make_async_copy(obufs[k], out_enc.at[pl.ds(0, _HG)],
                              osems[k]).wait()

    def sum_group(k):
        nbuf, obuf = nbufs[k], obufs[k]

        def sample_body(i, carry2):
            base = i * _K
            acc = tuple(nbuf[base, pl.ds(j * 16, 16)] for j in range(8))

            def kbody(kk, accs):
                return tuple(accs[j] + nbuf[base + kk, pl.ds(j * 16, 16)]
                             for j in range(8))

            acc = lax.fori_loop(1, _K, kbody, acc)
            for j in range(8):
                obuf[i, pl.ds(_D + j * 16, 16)] = acc[j]
                obuf[i, pl.ds(j * 16, 16)] = nbuf[_NROWS + i,
                                                  pl.ds(j * 16, 16)]
            return carry2

        lax.fori_loop(0, _HG, sample_body, 0)

    stage_idx(0, idx0).wait()
    stage_idx(1, idx1)
    for k in range(_NSLOT - 1):
        issue(jnp.int32(k), k)

    def ring_body(t, carry):
        for k in range(_NSLOT):
            hcons = _NSLOT * t + k
            wait_gather(k)

            @pl.when(t > 0)
            def _():
                wait_out(k)

            sum_group(k)
            pltpu.async_copy(
                obufs[k],
                out_enc.at[pl.ds((hbase + hcons) * _HG, _HG)], osems[k])

            hpre = hcons + (_NSLOT - 1)
            mod = lax.rem(hpre, _IBATCH)
            bb = lax.div(hpre, _IBATCH)

            @pl.when(jnp.logical_and(mod == 0, hpre < _HGPW))
            def _():
                pltpu.make_async_copy(
                    idx5_hbm.at[pl.ds(0, _IBATCH * _STRIDE)], idx0,
                    semI).wait()

            @pl.when(jnp.logical_and(mod == 2, bb + 1 <= _NBATCH - 1))
            def _():
                @pl.when(lax.rem(bb + 1, 2) == 0)
                def _():
                    stage_idx(bb + 1, idx0)

                @pl.when(lax.rem(bb + 1, 2) == 1)
                def _():
                    stage_idx(bb + 1, idx1)

            @pl.when(hpre < _HGPW)
            def _():
                issue(hpre, (k + _NSLOT - 1) % _NSLOT)
        return carry

    lax.fori_loop(0, _HGPW // _NSLOT, ring_body, 0)
    for k in range(_NSLOT):
        wait_out(k)

    pltpu.async_copy(relidx_hbm.at[pl.ds(w * _RCHUNKS, _RCHUNKS)],
                     relidx_v, semI).wait()
    pltpu.async_copy(rel_hbm.at[relidx_v.at[0]],
                     nb0.at[pl.ds(0, 128)], semR0)

    def rel_chunk(t, carry):
        @pl.when(t + 1 < _RCHUNKS)
        def _():
            @pl.when(lax.rem(t, 2) == 0)
            def _():
                pltpu.async_copy(rel_hbm.at[relidx_v.at[t + 1]],
                                 nb1.at[pl.ds(0, 128)], semR1)

            @pl.when(lax.rem(t, 2) == 1)
            def _():
                pltpu.async_copy(rel_hbm.at[relidx_v.at[t + 1]],
                                 nb0.at[pl.ds(0, 128)], semR0)

        @pl.when(lax.rem(t, 2) == 0)
        def _():
            pltpu.make_async_copy(rel_hbm.at[relidx_v.at[0]],
                                  nb0.at[pl.ds(0, 128)], semR0).wait()
            pltpu.sync_copy(nb0.at[pl.ds(0, 128)],
                            out_rel.at[pl.ds(w * _RPW + t * 128, 128)])

        @pl.when(lax.rem(t, 2) == 1)
        def _():
            pltpu.make_async_copy(rel_hbm.at[relidx_v.at[0]],
                                  nb1.at[pl.ds(0, 128)], semR1).wait()
            pltpu.sync_copy(nb1.at[pl.ds(0, 128)],
                            out_rel.at[pl.ds(w * _RPW + t * 128, 128)])
        return carry

    lax.fori_loop(0, _RCHUNKS, rel_chunk, 0)


def _sc_gather(idx5, relidx, node_emb, rel_emb):
    mesh = plsc.VectorSubcoreMesh(core_axis_name="c", subcore_axis_name="s")
    return pl.kernel(
        _sc_body,
        out_type=(
            jax.ShapeDtypeStruct((_NE, 2 * _D), jnp.float32),
            jax.ShapeDtypeStruct((_NS, _D), jnp.float32),
        ),
        mesh=mesh,
        scratch_types=[
            pltpu.VMEM((_IBATCH * _STRIDE,), jnp.int32),
            pltpu.VMEM((_IBATCH * _STRIDE,), jnp.int32),
            pltpu.VMEM((_GROWS, _D), jnp.float32),
            pltpu.VMEM((_GROWS, _D), jnp.float32),
            pltpu.VMEM((_GROWS, _D), jnp.float32),
            pltpu.VMEM((_GROWS, _D), jnp.float32),
            pltpu.VMEM((_HG, 2 * _D), jnp.float32),
            pltpu.VMEM((_HG, 2 * _D), jnp.float32),
            pltpu.VMEM((_HG, 2 * _D), jnp.float32),
            pltpu.VMEM((_HG, 2 * _D), jnp.float32),
            pltpu.VMEM((_RCHUNKS, 128), jnp.int32),
            pltpu.SemaphoreType.DMA,
            pltpu.SemaphoreType.DMA,
            pltpu.SemaphoreType.DMA,
            pltpu.SemaphoreType.DMA,
            pltpu.SemaphoreType.DMA,
            pltpu.SemaphoreType.DMA,
            pltpu.SemaphoreType.DMA,
            pltpu.SemaphoreType.DMA,
            pltpu.SemaphoreType.DMA,
            pltpu.SemaphoreType.DMA,
            pltpu.SemaphoreType.DMA,
        ],
    )(idx5, relidx, node_emb, rel_emb)


_BP = 2048
_NBLK = _P // _BP


def _tc_body(enc_sp, enc_sn, enc_dp, enc_dn,
             rel_p, rel_n, wf, bvec, out):
    i = pl.program_id(0)

    def enc(x):
        h = lax.dot_general(x[...], wf[...], (((1,), (0,)), ((), ())),
                            preferred_element_type=jnp.float32)
        return jnp.maximum(h + bvec[...], 0.0)

    hsp = enc(enc_sp)
    hsn = enc(enc_sn)
    hdp = enc(enc_dp)
    hdn = enc(enc_dn)
    dp = hsp + rel_p[...] - hdp
    dn = hsn + rel_n[...] - hdn
    sp = -jnp.sqrt(jnp.sum(dp * dp, axis=1) + 1e-12)
    sn = -jnp.sqrt(jnp.sum(dn * dn, axis=1) + 1e-12)
    part = jnp.sum(jnp.maximum(0.0, sn - sp + _MARGIN)) * (1.0 / _P)

    @pl.when(i == 0)
    def _():
        out[0, 0] = 0.0

    out[0, 0] += part


def _tc_dense(out_enc, out_rel, wf, bvec):
    enc_spec = lambda off: pl.BlockSpec((_BP, 2 * _D),
                                        lambda i, o=off: (i + o, 0))
    rel_spec = lambda off: pl.BlockSpec((_BP, _D),
                                        lambda i, o=off: (i + o, 0))
    loss = pl.pallas_call(
        _tc_body,
        grid=(_NBLK,),
        in_specs=[
            enc_spec(0), enc_spec(_NBLK), enc_spec(2 * _NBLK),
            enc_spec(3 * _NBLK),
            rel_spec(0), rel_spec(_NBLK),
            pl.BlockSpec((2 * _D, _D), lambda i: (0, 0)),
            pl.BlockSpec((1, _D), lambda i: (0, 0)),
        ],
        out_specs=pl.BlockSpec((1, 1), lambda i: (0, 0),
                               memory_space=pltpu.SMEM),
        out_shape=jax.ShapeDtypeStruct((1, 1), jnp.float32),
        compiler_params=pltpu.CompilerParams(
            dimension_semantics=("arbitrary",)),
    )(out_enc, out_enc, out_enc, out_enc,
      out_rel, out_rel, wf, bvec)
    return loss[0, 0]


def kernel(train_pos, train_neg, ngh_idx_src, ngh_idx_dst,
           node_emb, rel_emb, W, b):
    alls = jnp.concatenate([train_pos, train_neg], axis=0).astype(jnp.int32)
    src = alls[:, 0] % _NODE_NUM
    dst = alls[:, 1] % _NODE_NUM
    rel = alls[:, 2] % _REL_NUM

    selfidx = jnp.concatenate([src, dst]).reshape(_NHG, _HG)
    nghidx = jnp.concatenate(
        [ngh_idx_src, ngh_idx_dst], axis=0).astype(jnp.int32).reshape(
            _NHG, _NROWS)
    pad = jnp.zeros((_NHG, _STRIDE - _GROWS), dtype=jnp.int32)
    idx5 = jnp.concatenate([nghidx, selfidx, pad], axis=1).reshape(-1)
    relidx = rel.reshape(_NS // 128, 128)

    out_enc, out_rel = _sc_gather(idx5, relidx, node_emb, rel_emb)

    wf = jnp.concatenate([W[:_D], W[_D:] * (1.0 / _K)], axis=0)
    bvec = b.reshape(1, _D)
    return _tc_dense(out_enc, out_rel, wf, bvec)


# speedup vs baseline: 1.0114x; 1.0094x over previous
import jax
import jax.numpy as jnp
from jax import lax
from jax.experimental import pallas as pl
from jax.experimental.pallas import tpu as pltpu
from jax.experimental.pallas import tpu_sc as plsc

_NODE_NUM = 100000
_REL_NUM = 1000
_D = 128
_K = 20
_P = 16384
_NS = 2 * _P
_NE = 2 * _NS
_MARGIN = 1.0

_NW = 32
_HG = 8
_NHG = _NE // _HG
_HGPW = _NHG // _NW
_NROWS = _HG * _K
_GROWS = _NROWS + _HG
_STRIDE = 176
_IBATCH = 64
_NBATCH = _HGPW // _IBATCH
_NSLOT = 4
_RPW = _NS // _NW
_RCHUNKS = _RPW // 128


def _sc_body(idx5_hbm, relidx_hbm, node_hbm, rel_hbm,
             out_enc, out_rel,
             idx0, idx1, nb0, nb1, nb2, nb3, ob0, ob1, ob2, ob3, relidx_v,
             sg0, sg1, sg2, sg3, so0, so1, so2, so3, semI, semR0, semR1):
    c = lax.axis_index("c")
    s = lax.axis_index("s")
    w = s * 2 + c
    hbase = w * _HGPW
    ibase = w * (_HGPW * _STRIDE)

    nbufs = [nb0, nb1, nb2, nb3]
    obufs = [ob0, ob1, ob2, ob3]
    gsems = [sg0, sg1, sg2, sg3]
    osems = [so0, so1, so2, so3]

    def stage_idx(bi, ibuf):
        return pltpu.async_copy(
            idx5_hbm.at[pl.ds(ibase + bi * (_IBATCH * _STRIDE),
                              _IBATCH * _STRIDE)], ibuf, semI)

    def issue(H, k):
        off = lax.rem(H, _IBATCH) * _STRIDE
        par = lax.rem(lax.div(H, _IBATCH), 2)

        @pl.when(par == 0)
        def _():
            pltpu.async_copy(node_hbm.at[idx0.at[pl.ds(off, _GROWS)]],
                             nbufs[k].at[pl.ds(0, _GROWS)], gsems[k])

        @pl.when(par == 1)
        def _():
            pltpu.async_copy(node_hbm.at[idx1.at[pl.ds(off, _GROWS)]],
                             nbufs[k].at[pl.ds(0, _GROWS)], gsems[k])

    def wait_gather(k):
        pltpu.make_async_copy(node_hbm.at[idx0.at[pl.ds(0, _GROWS)]],
                              nbufs[k].at[pl.ds(0, _GROWS)], gsems[k]).wait()

    def wait_out(k):
        pltpu.make_async_copy(obufs[k], out_enc.at[pl.ds(0, _HG)],
                              osems[k]).wait()

    def sum_group(k):
        nbuf, obuf = nbufs[k], obufs[k]

        def sample_body(i, carry2):
            base = i * _K
            acc = tuple(nbuf[base, pl.ds(j * 16, 16)] for j in range(8))

            def kbody(kk, accs):
                accs = tuple(accs[j] + nbuf[base + 2 * kk + 1,
                                            pl.ds(j * 16, 16)]
                             for j in range(8))
                return tuple(accs[j] + nbuf[base + 2 * kk + 2,
                                            pl.ds(j * 16, 16)]
                             for j in range(8))

            acc = lax.fori_loop(0, 9, kbody, acc)
            acc = tuple(acc[j] + nbuf[base + 19, pl.ds(j * 16, 16)]
                        for j in range(8))
            for j in range(8):
                obuf[i, pl.ds(_D + j * 16, 16)] = acc[j]
                obuf[i, pl.ds(j * 16, 16)] = nbuf[_NROWS + i,
                                                  pl.ds(j * 16, 16)]
            return carry2

        lax.fori_loop(0, _HG, sample_body, 0)

    stage_idx(0, idx0).wait()
    stage_idx(1, idx1)
    for k in range(_NSLOT - 1):
        issue(jnp.int32(k), k)

    def ring_body(t, carry):
        for k in range(_NSLOT):
            hcons = _NSLOT * t + k
            wait_gather(k)

            hpre = hcons + (_NSLOT - 1)
            mod = lax.rem(hpre, _IBATCH)
            bb = lax.div(hpre, _IBATCH)

            @pl.when(jnp.logical_and(mod == 0, hpre < _HGPW))
            def _():
                pltpu.make_async_copy(
                    idx5_hbm.at[pl.ds(0, _IBATCH * _STRIDE)], idx0,
                    semI).wait()

            @pl.when(hpre < _HGPW)
            def _():
                issue(hpre, (k + _NSLOT - 1) % _NSLOT)

            @pl.when(jnp.logical_and(mod == 2, bb + 1 <= _NBATCH - 1))
            def _():
                @pl.when(lax.rem(bb + 1, 2) == 0)
                def _():
                    stage_idx(bb + 1, idx0)

                @pl.when(lax.rem(bb + 1, 2) == 1)
                def _():
                    stage_idx(bb + 1, idx1)

            @pl.when(t > 0)
            def _():
                wait_out(k)

            sum_group(k)
            pltpu.async_copy(
                obufs[k],
                out_enc.at[pl.ds((hbase + hcons) * _HG, _HG)], osems[k])
        return carry

    lax.fori_loop(0, _HGPW // _NSLOT, ring_body, 0)
    for k in range(_NSLOT):
        wait_out(k)

    pltpu.async_copy(relidx_hbm.at[pl.ds(w * _RCHUNKS, _RCHUNKS)],
                     relidx_v, semI).wait()
    pltpu.async_copy(rel_hbm.at[relidx_v.at[0]],
                     nb0.at[pl.ds(0, 128)], semR0)

    def rel_chunk(t, carry):
        @pl.when(t + 1 < _RCHUNKS)
        def _():
            @pl.when(lax.rem(t, 2) == 0)
            def _():
                pltpu.async_copy(rel_hbm.at[relidx_v.at[t + 1]],
                                 nb1.at[pl.ds(0, 128)], semR1)

            @pl.when(lax.rem(t, 2) == 1)
            def _():
                pltpu.async_copy(rel_hbm.at[relidx_v.at[t + 1]],
                                 nb0.at[pl.ds(0, 128)], semR0)

        @pl.when(lax.rem(t, 2) == 0)
        def _():
            pltpu.make_async_copy(rel_hbm.at[relidx_v.at[0]],
                                  nb0.at[pl.ds(0, 128)], semR0).wait()
            pltpu.sync_copy(nb0.at[pl.ds(0, 128)],
                            out_rel.at[pl.ds(w * _RPW + t * 128, 128)])

        @pl.when(lax.rem(t, 2) == 1)
        def _():
            pltpu.make_async_copy(rel_hbm.at[relidx_v.at[0]],
                                  nb1.at[pl.ds(0, 128)], semR1).wait()
            pltpu.sync_copy(nb1.at[pl.ds(0, 128)],
                            out_rel.at[pl.ds(w * _RPW + t * 128, 128)])
        return carry

    lax.fori_loop(0, _RCHUNKS, rel_chunk, 0)


def _sc_gather(idx5, relidx, node_emb, rel_emb):
    mesh = plsc.VectorSubcoreMesh(core_axis_name="c", subcore_axis_name="s")
    return pl.kernel(
        _sc_body,
        out_type=(
            jax.ShapeDtypeStruct((_NE, 2 * _D), jnp.float32),
            jax.ShapeDtypeStruct((_NS, _D), jnp.float32),
        ),
        mesh=mesh,
        scratch_types=[
            pltpu.VMEM((_IBATCH * _STRIDE,), jnp.int32),
            pltpu.VMEM((_IBATCH * _STRIDE,), jnp.int32),
            pltpu.VMEM((_GROWS, _D), jnp.float32),
            pltpu.VMEM((_GROWS, _D), jnp.float32),
            pltpu.VMEM((_GROWS, _D), jnp.float32),
            pltpu.VMEM((_GROWS, _D), jnp.float32),
            pltpu.VMEM((_HG, 2 * _D), jnp.float32),
            pltpu.VMEM((_HG, 2 * _D), jnp.float32),
            pltpu.VMEM((_HG, 2 * _D), jnp.float32),
            pltpu.VMEM((_HG, 2 * _D), jnp.float32),
            pltpu.VMEM((_RCHUNKS, 128), jnp.int32),
            pltpu.SemaphoreType.DMA,
            pltpu.SemaphoreType.DMA,
            pltpu.SemaphoreType.DMA,
            pltpu.SemaphoreType.DMA,
            pltpu.SemaphoreType.DMA,
            pltpu.SemaphoreType.DMA,
            pltpu.SemaphoreType.DMA,
            pltpu.SemaphoreType.DMA,
            pltpu.SemaphoreType.DMA,
            pltpu.SemaphoreType.DMA,
            pltpu.SemaphoreType.DMA,
        ],
    )(idx5, relidx, node_emb, rel_emb)


_BP = 2048
_NBLK = _P // _BP


def _tc_body(enc_sp, enc_sn, enc_dp, enc_dn,
             rel_p, rel_n, wf, bvec, out):
    i = pl.program_id(0)

    def enc(x):
        h = lax.dot_general(x[...], wf[...], (((1,), (0,)), ((), ())),
                            preferred_element_type=jnp.float32)
        return jnp.maximum(h + bvec[...], 0.0)

    hsp = enc(enc_sp)
    hsn = enc(enc_sn)
    hdp = enc(enc_dp)
    hdn = enc(enc_dn)
    dp = hsp + rel_p[...] - hdp
    dn = hsn + rel_n[...] - hdn
    sp = -jnp.sqrt(jnp.sum(dp * dp, axis=1) + 1e-12)
    sn = -jnp.sqrt(jnp.sum(dn * dn, axis=1) + 1e-12)
    part = jnp.sum(jnp.maximum(0.0, sn - sp + _MARGIN)) * (1.0 / _P)

    @pl.when(i == 0)
    def _():
        out[0, 0] = 0.0

    out[0, 0] += part


def _tc_dense(out_enc, out_rel, wf, bvec):
    enc_spec = lambda off: pl.BlockSpec((_BP, 2 * _D),
                                        lambda i, o=off: (i + o, 0))
    rel_spec = lambda off: pl.BlockSpec((_BP, _D),
                                        lambda i, o=off: (i + o, 0))
    loss = pl.pallas_call(
        _tc_body,
        grid=(_NBLK,),
        in_specs=[
            enc_spec(0), enc_spec(_NBLK), enc_spec(2 * _NBLK),
            enc_spec(3 * _NBLK),
            rel_spec(0), rel_spec(_NBLK),
            pl.BlockSpec((2 * _D, _D), lambda i: (0, 0)),
            pl.BlockSpec((1, _D), lambda i: (0, 0)),
        ],
        out_specs=pl.BlockSpec((1, 1), lambda i: (0, 0),
                               memory_space=pltpu.SMEM),
        out_shape=jax.ShapeDtypeStruct((1, 1), jnp.float32),
        compiler_params=pltpu.CompilerParams(
            dimension_semantics=("arbitrary",)),
    )(out_enc, out_enc, out_enc, out_enc,
      out_rel, out_rel, wf, bvec)
    return loss[0, 0]


def kernel(train_pos, train_neg, ngh_idx_src, ngh_idx_dst,
           node_emb, rel_emb, W, b):
    alls = jnp.concatenate([train_pos, train_neg], axis=0).astype(jnp.int32)
    src = alls[:, 0] % _NODE_NUM
    dst = alls[:, 1] % _NODE_NUM
    rel = alls[:, 2] % _REL_NUM

    selfidx = jnp.concatenate([src, dst]).reshape(_NHG, _HG)
    nghidx = jnp.concatenate(
        [ngh_idx_src, ngh_idx_dst], axis=0).astype(jnp.int32).reshape(
            _NHG, _NROWS)
    pad = jnp.zeros((_NHG, _STRIDE - _GROWS), dtype=jnp.int32)
    idx5 = jnp.concatenate([nghidx, selfidx, pad], axis=1).reshape(-1)
    relidx = rel.reshape(_NS // 128, 128)

    out_enc, out_rel = _sc_gather(idx5, relidx, node_emb, rel_emb)

    wf = jnp.concatenate([W[:_D], W[_D:] * (1.0 / _K)], axis=0)
    bvec = b.reshape(1, _D)
    return _tc_dense(out_enc, out_rel, wf, bvec)
